# baseline (speedup 1.0000x reference)
import functools
import jax
import jax.numpy as jnp
from jax import lax
from jax.experimental import pallas as pl
from jax.experimental.pallas import tpu as pltpu
from jax.experimental.pallas import tpu_sc as plsc

N = 10000
E = 320000
D = 128
G = 64
BN_EPS = 1e-5

NC, NS = 2, 16
NW = NC * NS
CHUNK = 128
C0 = 64
C1 = 93
SEG = 64
MAXC = 192
E_CAP0 = NS * C0 * CHUNK
E_TOT = NS * (C0 + C1) * CHUNK
NP = 10112
NP0 = N
RPT = NP // NS

_mesh = plsc.VectorSubcoreMesh(core_axis_name="c", subcore_axis_name="s")


def _zero_tile_buf(buf, nrows, ncols):
    zv = jnp.zeros((16,), jnp.float32)

    def body(r, _):
        for c in range(ncols // 16):
            buf[r, pl.ds(c * 16, 16)] = zv
        return 0

    lax.fori_loop(0, nrows, body, 0)


def _zero_acc_slice(buf, acc_sh, base):
    done = 0
    while done < RPT:
        nr = min(CHUNK, RPT - done)
        pltpu.sync_copy(buf.at[pl.ds(0, nr)],
                        acc_sh.at[pl.ds(base + done, nr)])
        done += nr


@functools.partial(
    pl.kernel,
    out_type=jax.ShapeDtypeStruct((NC, NP, D), jnp.float32),
    mesh=_mesh,
    scratch_types=[
        pltpu.VMEM_SHARED((NP, D), jnp.float32),
        pltpu.VMEM((SEG, CHUNK), jnp.int32),
        pltpu.VMEM((CHUNK, D), jnp.float32),
        pltpu.SemaphoreType.DMA,
    ],
)
def _sc_deg(dst_hbm, out_hbm, acc_sh, dst_v, buf_v, sem):
    cid = lax.axis_index("c")
    sid = lax.axis_index("s")
    wid = cid * NS + sid
    myc = jnp.where(cid == 0, C0, C1)

    _zero_tile_buf(buf_v, CHUNK, D)
    base = sid * RPT
    _zero_acc_slice(buf_v, acc_sh, base)

    ones = jnp.ones((16,), jnp.float32)

    def fill(r, _):
        for c in range(D // 16):
            buf_v[r, pl.ds(c * 16, 16)] = ones
        return 0

    lax.fori_loop(0, CHUNK, fill, 0)
    plsc.subcore_barrier()

    DEPTH = 4

    def issue(j):
        pltpu.async_copy(buf_v, acc_sh.at[dst_v.at[j]], sem, add=True)

    def drain(j):
        pltpu.make_async_copy(buf_v, acc_sh.at[dst_v.at[j]], sem).wait()

    def seg_body(s, _):
        nj = jnp.minimum(SEG, myc - s * SEG)

        pltpu.sync_copy(dst_hbm.at[wid].at[pl.ds(s * SEG, SEG)], dst_v)

        def prime(j, _):
            issue(j)
            return 0

        np_ = jnp.minimum(nj, DEPTH)
        lax.fori_loop(0, np_, prime, 0)

        def chunk(j, _):
            issue(j)
            drain(j - DEPTH)
            return 0

        lax.fori_loop(np_, nj, chunk, 0)

        def dr(j, _):
            drain(j)
            return 0

        lax.fori_loop(jnp.maximum(nj - DEPTH, 0), nj, dr, 0)
        return 0

    nseg = (myc + SEG - 1) // SEG
    lax.fori_loop(0, nseg, seg_body, 0)

    plsc.subcore_barrier()
    pltpu.sync_copy(acc_sh.at[pl.ds(base, RPT)],
                    out_hbm.at[cid].at[pl.ds(base, RPT)])


@functools.partial(
    pl.kernel,
    out_type=jax.ShapeDtypeStruct((NC, NP, D), jnp.float32),
    mesh=_mesh,
    scratch_types=[
        pltpu.VMEM_SHARED((NP, D), jnp.float32),
        pltpu.VMEM((SEG, CHUNK), jnp.int32),
        pltpu.VMEM((SEG, CHUNK), jnp.int32),
        pltpu.VMEM((CHUNK, D), jnp.float32),
        pltpu.SemaphoreType.DMA,
    ],
)
def _sc_prop(hp_hbm, src_hbm, dst_hbm, out_hbm,
             acc_sh, src_v, dst_v, rows_v, sem):
    cid = lax.axis_index("c")
    sid = lax.axis_index("s")
    wid = cid * NS + sid
    myc = jnp.where(cid == 0, C0, C1)

    _zero_tile_buf(rows_v, CHUNK, D)
    base = sid * RPT
    _zero_acc_slice(rows_v, acc_sh, base)
    plsc.subcore_barrier()

    def seg_body(s, _):
        nj = jnp.minimum(SEG, myc - s * SEG)
        pltpu.sync_copy(src_hbm.at[wid].at[pl.ds(s * SEG, SEG)], src_v)
        pltpu.sync_copy(dst_hbm.at[wid].at[pl.ds(s * SEG, SEG)], dst_v)

        def chunk(j, _):
            pltpu.async_copy(hp_hbm.at[src_v.at[j]], rows_v, sem).wait()
            pltpu.sync_copy(rows_v, acc_sh.at[dst_v.at[j]], add=True)
            return 0

        lax.fori_loop(0, nj, chunk, 0)
        return 0

    nseg = (myc + SEG - 1) // SEG
    lax.fori_loop(0, nseg, seg_body, 0)

    plsc.subcore_barrier()
    pltpu.sync_copy(acc_sh.at[pl.ds(base, RPT)],
                    out_hbm.at[cid].at[pl.ds(base, RPT)])


def _tc_pre_body(deg2_ref, x_ref, w1_ref, dinv_ref, h1p_ref):
    deg = deg2_ref[0] + deg2_ref[1]
    dinv_b = jax.lax.rsqrt(deg + 1.0)
    dinv_ref[...] = dinv_b
    h = jnp.dot(x_ref[...], w1_ref[...].T, preferred_element_type=jnp.float32)
    h1p_ref[...] = dinv_b * h


def _tc_mid_body(acc_ref, hp_ref, dinv_ref, b_ref, g_ref, be_ref, wn_ref,
                 out_ref):
    dinv_b = dinv_ref[...]
    z = dinv_b * (acc_ref[0] + acc_ref[1] + hp_ref[...]) + b_ref[...]
    z = z * (g_ref[...] * jax.lax.rsqrt(jnp.float32(1.0 + BN_EPS))) + be_ref[...]
    z = jnp.maximum(z, 0.0)
    h = jnp.dot(z, wn_ref[...].T, preferred_element_type=jnp.float32)
    rows = lax.broadcasted_iota(jnp.int32, (NP, D), 0)
    out_ref[...] = jnp.where(rows < N, dinv_b * h, 0.0)


def _tc_final_body(acc_ref, hp_ref, dinv_ref, b_ref, batch_ref,
                   lw1_ref, lb1_ref, lw2b_ref, lb2b_ref, out_ref):
    h3 = dinv_ref[...] * (acc_ref[0] + acc_ref[1] + hp_ref[...]) + b_ref[...]
    h3 = h3[:N]
    seg = lax.broadcasted_iota(jnp.int32, (G, N), 0)
    bat = jnp.broadcast_to(batch_ref[...][None, :], (G, N))
    mask = (seg == bat).astype(jnp.float32)
    sums = jnp.dot(mask, h3, preferred_element_type=jnp.float32)
    cnt = jnp.dot(mask, jnp.full((N, D), 1.0, jnp.float32),
                  preferred_element_type=jnp.float32)
    pooled = sums / jnp.maximum(cnt, 1.0)
    hm = jnp.dot(pooled, lw1_ref[...].T, preferred_element_type=jnp.float32)
    hm = jnp.maximum(hm + lb1_ref[...], 0.0)
    o = jnp.dot(hm, lw2b_ref[...], preferred_element_type=jnp.float32)
    out_ref[...] = o + lb2b_ref[...]


_tc_pre = pl.pallas_call(
    _tc_pre_body,
    out_shape=(jax.ShapeDtypeStruct((NP, D), jnp.float32),
               jax.ShapeDtypeStruct((NP, D), jnp.float32)),
)

_tc_mid = pl.pallas_call(
    _tc_mid_body,
    out_shape=jax.ShapeDtypeStruct((NP, D), jnp.float32),
)

_tc_final = pl.pallas_call(
    _tc_final_body,
    out_shape=jax.ShapeDtypeStruct((G, D), jnp.float32),
)


def _edge_layout(idx):
    ec = jnp.concatenate([idx, jnp.full((E_TOT - E,), NP0, jnp.int32)])
    p0 = ec[:E_CAP0].reshape(NS, C0, CHUNK)
    p0 = jnp.concatenate(
        [p0, jnp.full((NS, MAXC - C0, CHUNK), NP0, jnp.int32)], axis=1)
    p1 = ec[E_CAP0:].reshape(NS, C1, CHUNK)
    p1 = jnp.concatenate(
        [p1, jnp.full((NS, MAXC - C1, CHUNK), NP0, jnp.int32)], axis=1)
    return jnp.concatenate([p0, p1], axis=0)


def kernel(x, edge_index, batch, W1, b1, g1, be1, W2, b2, g2, be2, W3, b3,
           lw1, lb1, lw2, lb2):
    src_p = _edge_layout(edge_index[0])
    dst_p = _edge_layout(edge_index[1])

    x_p = jnp.zeros((NP, D), jnp.float32).at[:N].set(x)

    deg2 = _sc_deg(dst_p)
    dinv_b, h1p = _tc_pre(deg2, x_p, W1)
    acc1 = _sc_prop(h1p, src_p, dst_p)
    h2p = _tc_mid(acc1, h1p, dinv_b, b1, g1, be1, W2)
    acc2 = _sc_prop(h2p, src_p, dst_p)
    h3p = _tc_mid(acc2, h2p, dinv_b, b2, g2, be2, W3)
    acc3 = _sc_prop(h3p, src_p, dst_p)
    lw2b = jnp.broadcast_to(lw2.T, (G, D))
    lb2b = jnp.broadcast_to(lb2, (D,))
    out128 = _tc_final(acc3, h3p, dinv_b, b3, batch, lw1, lb1, lw2b, lb2b)
    return out128[:, :1]


# speedup vs baseline: 1.1858x; 1.1858x over previous
import functools
import jax
import jax.numpy as jnp
from jax import lax
from jax.experimental import pallas as pl
from jax.experimental.pallas import tpu as pltpu
from jax.experimental.pallas import tpu_sc as plsc

N = 10000
E = 320000
D = 128
G = 64
BN_EPS = 1e-5

NC, NS = 2, 16
NW = NC * NS
CHUNK = 128
C0 = 66
C1 = 91
SEG = 32
MAXC = 96
E_CAP0 = NS * C0 * CHUNK
E_TOT = NS * (C0 + C1) * CHUNK
NP = 10112
NP0 = N
RPT = NP // NS

_mesh = plsc.VectorSubcoreMesh(core_axis_name="c", subcore_axis_name="s")


def _zero_tile_buf(buf, nrows, ncols):
    zv = jnp.zeros((16,), jnp.float32)

    def body(r, _):
        for c in range(ncols // 16):
            buf[r, pl.ds(c * 16, 16)] = zv
        return 0

    lax.fori_loop(0, nrows, body, 0)


def _zero_acc_slice(buf, acc_sh, base):
    done = 0
    while done < RPT:
        nr = min(CHUNK, RPT - done)
        pltpu.sync_copy(buf.at[pl.ds(0, nr)],
                        acc_sh.at[pl.ds(base + done, nr)])
        done += nr


@functools.partial(
    pl.kernel,
    out_type=jax.ShapeDtypeStruct((NC, NP, D), jnp.float32),
    mesh=_mesh,
    scratch_types=[
        pltpu.VMEM_SHARED((NP, D), jnp.float32),
        pltpu.VMEM((SEG, CHUNK), jnp.int32),
        pltpu.VMEM((CHUNK, D), jnp.float32),
        pltpu.SemaphoreType.DMA,
    ],
)
def _sc_deg(dst_hbm, out_hbm, acc_sh, dst_v, buf_v, sem):
    cid = lax.axis_index("c")
    sid = lax.axis_index("s")
    wid = cid * NS + sid
    myc = jnp.where(cid == 0, C0, C1)

    _zero_tile_buf(buf_v, CHUNK, D)
    base = sid * RPT
    _zero_acc_slice(buf_v, acc_sh, base)

    ones = jnp.ones((16,), jnp.float32)

    def fill(r, _):
        for c in range(D // 16):
            buf_v[r, pl.ds(c * 16, 16)] = ones
        return 0

    lax.fori_loop(0, CHUNK, fill, 0)
    plsc.subcore_barrier()

    DEPTH = 4

    def issue(j):
        pltpu.async_copy(buf_v, acc_sh.at[dst_v.at[j]], sem, add=True)

    def drain(j):
        pltpu.make_async_copy(buf_v, acc_sh.at[dst_v.at[j]], sem).wait()

    def seg_body(s, _):
        nj = jnp.minimum(SEG, myc - s * SEG)

        pltpu.sync_copy(dst_hbm.at[wid].at[pl.ds(s * SEG, SEG)], dst_v)

        def prime(j, _):
            issue(j)
            return 0

        np_ = jnp.minimum(nj, DEPTH)
        lax.fori_loop(0, np_, prime, 0)

        def chunk(j, _):
            issue(j)
            drain(j - DEPTH)
            return 0

        lax.fori_loop(np_, nj, chunk, 0)

        def dr(j, _):
            drain(j)
            return 0

        lax.fori_loop(jnp.maximum(nj - DEPTH, 0), nj, dr, 0)
        return 0

    nseg = (myc + SEG - 1) // SEG
    lax.fori_loop(0, nseg, seg_body, 0)

    plsc.subcore_barrier()
    pltpu.sync_copy(acc_sh.at[pl.ds(base, RPT)],
                    out_hbm.at[cid].at[pl.ds(base, RPT)])


@functools.partial(
    pl.kernel,
    out_type=jax.ShapeDtypeStruct((NC, NP, D), jnp.float32),
    mesh=_mesh,
    scratch_types=[
        pltpu.VMEM_SHARED((NP, D), jnp.float32),
        pltpu.VMEM((SEG, CHUNK), jnp.int32),
        pltpu.VMEM((SEG, CHUNK), jnp.int32),
        pltpu.VMEM((CHUNK, D), jnp.float32),
        pltpu.VMEM((CHUNK, D), jnp.float32),
        pltpu.SemaphoreType.DMA,
        pltpu.SemaphoreType.DMA,
        pltpu.SemaphoreType.DMA,
    ],
)
def _sc_prop(hp_hbm, src_hbm, dst_hbm, out_hbm,
             acc_sh, src_v, dst_v, buf0, buf1, gsem, s0, s1):
    cid = lax.axis_index("c")
    sid = lax.axis_index("s")
    wid = cid * NS + sid
    myc = jnp.where(cid == 0, C0, C1)
    bufs = (buf0, buf1)
    ssem = (s0, s1)

    _zero_tile_buf(buf0, CHUNK, D)
    base = sid * RPT
    _zero_acc_slice(buf0, acc_sh, base)
    plsc.subcore_barrier()

    def gather(j, b):
        pltpu.async_copy(hp_hbm.at[src_v.at[j]], bufs[b], gsem).wait()

    def s_issue(j, b):
        pltpu.async_copy(bufs[b], acc_sh.at[dst_v.at[j]], ssem[b], add=True)

    def s_wait(j, b):
        pltpu.make_async_copy(bufs[b], acc_sh.at[dst_v.at[j]], ssem[b]).wait()

    def seg_body(s, _):
        nj = jnp.minimum(SEG, myc - s * SEG)
        pltpu.sync_copy(src_hbm.at[wid].at[pl.ds(s * SEG, SEG)], src_v)
        pltpu.sync_copy(dst_hbm.at[wid].at[pl.ds(s * SEG, SEG)], dst_v)

        def pair(p, _):
            j0 = 2 * p
            j1 = j0 + 1

            @pl.when(p > 0)
            def _():
                s_wait(j0 - 2, 0)

            gather(j0, 0)
            s_issue(j0, 0)

            @pl.when(j1 < nj)
            def _():
                @pl.when(p > 0)
                def _():
                    s_wait(j1 - 2, 1)

                gather(j1, 1)
                s_issue(j1, 1)

            return 0

        npair = (nj + 1) // 2
        lax.fori_loop(0, npair, pair, 0)

        s_wait(0, 0)

        @pl.when(nj >= 2)
        def _():
            s_wait(0, 1)

        return 0

    nseg = (myc + SEG - 1) // SEG
    lax.fori_loop(0, nseg, seg_body, 0)

    plsc.subcore_barrier()
    pltpu.sync_copy(acc_sh.at[pl.ds(base, RPT)],
                    out_hbm.at[cid].at[pl.ds(base, RPT)])


def _tc_pre_body(deg2_ref, x_ref, w1_ref, dinv_ref, h1p_ref):
    deg = deg2_ref[0] + deg2_ref[1]
    dinv_b = jax.lax.rsqrt(deg + 1.0)
    dinv_ref[...] = dinv_b
    h = jnp.dot(x_ref[...], w1_ref[...].T, preferred_element_type=jnp.float32)
    h1p_ref[...] = dinv_b * h


def _tc_mid_body(acc_ref, hp_ref, dinv_ref, b_ref, g_ref, be_ref, wn_ref,
                 out_ref):
    dinv_b = dinv_ref[...]
    z = dinv_b * (acc_ref[0] + acc_ref[1] + hp_ref[...]) + b_ref[...]
    z = z * (g_ref[...] * jax.lax.rsqrt(jnp.float32(1.0 + BN_EPS))) + be_ref[...]
    z = jnp.maximum(z, 0.0)
    h = jnp.dot(z, wn_ref[...].T, preferred_element_type=jnp.float32)
    rows = lax.broadcasted_iota(jnp.int32, (NP, D), 0)
    out_ref[...] = jnp.where(rows < N, dinv_b * h, 0.0)


def _tc_final_body(acc_ref, hp_ref, dinv_ref, b_ref, batch_ref,
                   lw1_ref, lb1_ref, lw2b_ref, lb2b_ref, out_ref):
    h3 = dinv_ref[...] * (acc_ref[0] + acc_ref[1] + hp_ref[...]) + b_ref[...]
    h3 = h3[:N]
    seg = lax.broadcasted_iota(jnp.int32, (G, N), 0)
    bat = jnp.broadcast_to(batch_ref[...][None, :], (G, N))
    mask = (seg == bat).astype(jnp.float32)
    sums = jnp.dot(mask, h3, preferred_element_type=jnp.float32)
    cnt = jnp.dot(mask, jnp.full((N, D), 1.0, jnp.float32),
                  preferred_element_type=jnp.float32)
    pooled = sums / jnp.maximum(cnt, 1.0)
    hm = jnp.dot(pooled, lw1_ref[...].T, preferred_element_type=jnp.float32)
    hm = jnp.maximum(hm + lb1_ref[...], 0.0)
    o = jnp.dot(hm, lw2b_ref[...], preferred_element_type=jnp.float32)
    out_ref[...] = o + lb2b_ref[...]


_tc_pre = pl.pallas_call(
    _tc_pre_body,
    out_shape=(jax.ShapeDtypeStruct((NP, D), jnp.float32),
               jax.ShapeDtypeStruct((NP, D), jnp.float32)),
)

_tc_mid = pl.pallas_call(
    _tc_mid_body,
    out_shape=jax.ShapeDtypeStruct((NP, D), jnp.float32),
)

_tc_final = pl.pallas_call(
    _tc_final_body,
    out_shape=jax.ShapeDtypeStruct((G, D), jnp.float32),
)


def _edge_layout(idx):
    ec = jnp.concatenate([idx, jnp.full((E_TOT - E,), NP0, jnp.int32)])
    p0 = ec[:E_CAP0].reshape(NS, C0, CHUNK)
    p0 = jnp.concatenate(
        [p0, jnp.full((NS, MAXC - C0, CHUNK), NP0, jnp.int32)], axis=1)
    p1 = ec[E_CAP0:].reshape(NS, C1, CHUNK)
    p1 = jnp.concatenate(
        [p1, jnp.full((NS, MAXC - C1, CHUNK), NP0, jnp.int32)], axis=1)
    return jnp.concatenate([p0, p1], axis=0)


def kernel(x, edge_index, batch, W1, b1, g1, be1, W2, b2, g2, be2, W3, b3,
           lw1, lb1, lw2, lb2):
    src_p = _edge_layout(edge_index[0])
    dst_p = _edge_layout(edge_index[1])

    x_p = jnp.zeros((NP, D), jnp.float32).at[:N].set(x)

    deg2 = _sc_deg(dst_p)
    dinv_b, h1p = _tc_pre(deg2, x_p, W1)
    acc1 = _sc_prop(h1p, src_p, dst_p)
    h2p = _tc_mid(acc1, h1p, dinv_b, b1, g1, be1, W2)
    acc2 = _sc_prop(h2p, src_p, dst_p)
    h3p = _tc_mid(acc2, h2p, dinv_b, b2, g2, be2, W3)
    acc3 = _sc_prop(h3p, src_p, dst_p)
    lw2b = jnp.broadcast_to(lw2.T, (G, D))
    lb2b = jnp.broadcast_to(lb2, (D,))
    out128 = _tc_final(acc3, h3p, dinv_b, b3, batch, lw1, lb1, lw2b, lb2b)
    return out128[:, :1]


# speedup vs baseline: 1.2343x; 1.0410x over previous
import functools
import jax
import jax.numpy as jnp
from jax import lax
from jax.experimental import pallas as pl
from jax.experimental.pallas import tpu as pltpu
from jax.experimental.pallas import tpu_sc as plsc

N = 10000
E = 320000
D = 128
G = 64
BN_EPS = 1e-5

NC, NS = 2, 16
NW = NC * NS
CHUNK = 128
C0 = 71
C1 = 86
SEG = 32
MAXC = 96
E_CAP0 = NS * C0 * CHUNK
E_TOT = NS * (C0 + C1) * CHUNK
NP = 10112
NP0 = N
RPT = NP // NS

_mesh = plsc.VectorSubcoreMesh(core_axis_name="c", subcore_axis_name="s")


def _zero_tile_buf(buf, nrows, ncols):
    zv = jnp.zeros((16,), jnp.float32)

    def body(r, _):
        for c in range(ncols // 16):
            buf[r, pl.ds(c * 16, 16)] = zv
        return 0

    lax.fori_loop(0, nrows, body, 0)


def _zero_acc_slice(buf, acc_sh, base):
    done = 0
    while done < RPT:
        nr = min(CHUNK, RPT - done)
        pltpu.sync_copy(buf.at[pl.ds(0, nr)],
                        acc_sh.at[pl.ds(base + done, nr)])
        done += nr


@functools.partial(
    pl.kernel,
    out_type=jax.ShapeDtypeStruct((NC, NP, D), jnp.float32),
    mesh=_mesh,
    scratch_types=[
        pltpu.VMEM_SHARED((NP, D), jnp.float32),
        pltpu.VMEM((SEG, CHUNK), jnp.int32),
        pltpu.VMEM((CHUNK, D), jnp.float32),
        pltpu.SemaphoreType.DMA,
    ],
)
def _sc_deg(dst_hbm, out_hbm, acc_sh, dst_v, buf_v, sem):
    cid = lax.axis_index("c")
    sid = lax.axis_index("s")
    wid = cid * NS + sid
    myc = jnp.where(cid == 0, C0, C1)

    _zero_tile_buf(buf_v, CHUNK, D)
    base = sid * RPT
    _zero_acc_slice(buf_v, acc_sh, base)

    ones = jnp.ones((16,), jnp.float32)

    def fill(r, _):
        for c in range(D // 16):
            buf_v[r, pl.ds(c * 16, 16)] = ones
        return 0

    lax.fori_loop(0, CHUNK, fill, 0)
    plsc.subcore_barrier()

    DEPTH = 4

    def issue(j):
        pltpu.async_copy(buf_v, acc_sh.at[dst_v.at[j]], sem, add=True)

    def drain(j):
        pltpu.make_async_copy(buf_v, acc_sh.at[dst_v.at[j]], sem).wait()

    def seg_body(s, _):
        nj = jnp.minimum(SEG, myc - s * SEG)

        pltpu.sync_copy(dst_hbm.at[wid].at[pl.ds(s * SEG, SEG)], dst_v)

        def prime(j, _):
            issue(j)
            return 0

        np_ = jnp.minimum(nj, DEPTH)
        lax.fori_loop(0, np_, prime, 0)

        def chunk(j, _):
            issue(j)
            drain(j - DEPTH)
            return 0

        lax.fori_loop(np_, nj, chunk, 0)

        def dr(j, _):
            drain(j)
            return 0

        lax.fori_loop(jnp.maximum(nj - DEPTH, 0), nj, dr, 0)
        return 0

    nseg = (myc + SEG - 1) // SEG
    lax.fori_loop(0, nseg, seg_body, 0)

    plsc.subcore_barrier()
    pltpu.sync_copy(acc_sh.at[pl.ds(base, RPT)],
                    out_hbm.at[cid].at[pl.ds(base, RPT)])


@functools.partial(
    pl.kernel,
    out_type=jax.ShapeDtypeStruct((NC, NP, D), jnp.float32),
    mesh=_mesh,
    scratch_types=[
        pltpu.VMEM_SHARED((NP, D), jnp.float32),
        pltpu.VMEM((SEG, CHUNK), jnp.int32),
        pltpu.VMEM((SEG, CHUNK), jnp.int32),
        pltpu.VMEM((CHUNK, D), jnp.float32),
        pltpu.VMEM((CHUNK, D), jnp.float32),
        pltpu.SemaphoreType.DMA,
        pltpu.SemaphoreType.DMA,
        pltpu.SemaphoreType.DMA,
    ],
)
def _sc_prop(hp_hbm, src_hbm, dst_hbm, out_hbm,
             acc_sh, src_v, dst_v, buf0, buf1, gsem, s0, s1):
    cid = lax.axis_index("c")
    sid = lax.axis_index("s")
    wid = cid * NS + sid
    myc = jnp.where(cid == 0, C0, C1)
    bufs = (buf0, buf1)
    ssem = (s0, s1)

    _zero_tile_buf(buf0, CHUNK, D)
    base = sid * RPT
    _zero_acc_slice(buf0, acc_sh, base)
    plsc.subcore_barrier()

    def gather(j, b):
        pltpu.async_copy(hp_hbm.at[src_v.at[j]], bufs[b], gsem).wait()

    def s_issue(j, b):
        pltpu.async_copy(bufs[b], acc_sh.at[dst_v.at[j]], ssem[b], add=True)

    def s_wait(j, b):
        pltpu.make_async_copy(bufs[b], acc_sh.at[dst_v.at[j]], ssem[b]).wait()

    def seg_body(s, _):
        nj = jnp.minimum(SEG, myc - s * SEG)
        pltpu.sync_copy(src_hbm.at[wid].at[pl.ds(s * SEG, SEG)], src_v)
        pltpu.sync_copy(dst_hbm.at[wid].at[pl.ds(s * SEG, SEG)], dst_v)

        def pair(p, _):
            j0 = 2 * p
            j1 = j0 + 1

            @pl.when(p > 0)
            def _():
                s_wait(j0 - 2, 0)

            gather(j0, 0)
            s_issue(j0, 0)

            @pl.when(j1 < nj)
            def _():
                @pl.when(p > 0)
                def _():
                    s_wait(j1 - 2, 1)

                gather(j1, 1)
                s_issue(j1, 1)

            return 0

        npair = (nj + 1) // 2
        lax.fori_loop(0, npair, pair, 0)

        s_wait(0, 0)

        @pl.when(nj >= 2)
        def _():
            s_wait(0, 1)

        return 0

    nseg = (myc + SEG - 1) // SEG
    lax.fori_loop(0, nseg, seg_body, 0)

    plsc.subcore_barrier()
    pltpu.sync_copy(acc_sh.at[pl.ds(base, RPT)],
                    out_hbm.at[cid].at[pl.ds(base, RPT)])


def _tc_pre_body(deg2_ref, x_ref, w1_ref, dinv_ref, h1p_ref):
    deg = deg2_ref[0] + deg2_ref[1]
    dinv_b = jax.lax.rsqrt(deg + 1.0)
    dinv_ref[...] = dinv_b
    h = jnp.dot(x_ref[...], w1_ref[...].T, preferred_element_type=jnp.float32)
    h1p_ref[...] = dinv_b * h


def _tc_mid_body(acc_ref, hp_ref, dinv_ref, b_ref, g_ref, be_ref, wn_ref,
                 out_ref):
    dinv_b = dinv_ref[...]
    z = dinv_b * (acc_ref[0] + acc_ref[1] + hp_ref[...]) + b_ref[...]
    z = z * (g_ref[...] * jax.lax.rsqrt(jnp.float32(1.0 + BN_EPS))) + be_ref[...]
    z = jnp.maximum(z, 0.0)
    h = jnp.dot(z, wn_ref[...].T, preferred_element_type=jnp.float32)
    rows = lax.broadcasted_iota(jnp.int32, (NP, D), 0)
    out_ref[...] = jnp.where(rows < N, dinv_b * h, 0.0)


def _tc_final_body(acc_ref, hp_ref, dinv_ref, b_ref, batch_ref,
                   lw1_ref, lb1_ref, lw2b_ref, lb2b_ref, out_ref):
    h3 = dinv_ref[...] * (acc_ref[0] + acc_ref[1] + hp_ref[...]) + b_ref[...]
    h3 = h3[:N]
    seg = lax.broadcasted_iota(jnp.int32, (G, N), 0)
    bat = jnp.broadcast_to(batch_ref[...][None, :], (G, N))
    mask = (seg == bat).astype(jnp.float32)
    sums = jnp.dot(mask, h3, preferred_element_type=jnp.float32)
    cnt = jnp.dot(mask, jnp.full((N, D), 1.0, jnp.float32),
                  preferred_element_type=jnp.float32)
    pooled = sums / jnp.maximum(cnt, 1.0)
    hm = jnp.dot(pooled, lw1_ref[...].T, preferred_element_type=jnp.float32)
    hm = jnp.maximum(hm + lb1_ref[...], 0.0)
    o = jnp.dot(hm, lw2b_ref[...], preferred_element_type=jnp.float32)
    out_ref[...] = o + lb2b_ref[...]


_tc_pre = pl.pallas_call(
    _tc_pre_body,
    out_shape=(jax.ShapeDtypeStruct((NP, D), jnp.float32),
               jax.ShapeDtypeStruct((NP, D), jnp.float32)),
)

_tc_mid = pl.pallas_call(
    _tc_mid_body,
    out_shape=jax.ShapeDtypeStruct((NP, D), jnp.float32),
)

_tc_final = pl.pallas_call(
    _tc_final_body,
    out_shape=jax.ShapeDtypeStruct((G, D), jnp.float32),
)


def _edge_layout(idx):
    ec = jnp.concatenate([idx, jnp.full((E_TOT - E,), NP0, jnp.int32)])
    p0 = ec[:E_CAP0].reshape(NS, C0, CHUNK)
    p0 = jnp.concatenate(
        [p0, jnp.full((NS, MAXC - C0, CHUNK), NP0, jnp.int32)], axis=1)
    p1 = ec[E_CAP0:].reshape(NS, C1, CHUNK)
    p1 = jnp.concatenate(
        [p1, jnp.full((NS, MAXC - C1, CHUNK), NP0, jnp.int32)], axis=1)
    return jnp.concatenate([p0, p1], axis=0)


def kernel(x, edge_index, batch, W1, b1, g1, be1, W2, b2, g2, be2, W3, b3,
           lw1, lb1, lw2, lb2):
    src_p = _edge_layout(edge_index[0])
    dst_p = _edge_layout(edge_index[1])

    x_p = jnp.zeros((NP, D), jnp.float32).at[:N].set(x)

    deg2 = _sc_deg(dst_p)
    dinv_b, h1p = _tc_pre(deg2, x_p, W1)
    acc1 = _sc_prop(h1p, src_p, dst_p)
    h2p = _tc_mid(acc1, h1p, dinv_b, b1, g1, be1, W2)
    acc2 = _sc_prop(h2p, src_p, dst_p)
    h3p = _tc_mid(acc2, h2p, dinv_b, b2, g2, be2, W3)
    acc3 = _sc_prop(h3p, src_p, dst_p)
    lw2b = jnp.broadcast_to(lw2.T, (G, D))
    lb2b = jnp.broadcast_to(lb2, (D,))
    out128 = _tc_final(acc3, h3p, dinv_b, b3, batch, lw1, lb1, lw2b, lb2b)
    return out128[:, :1]


# speedup vs baseline: 1.2563x; 1.0178x over previous
import functools
import jax
import jax.numpy as jnp
from jax import lax
from jax.experimental import pallas as pl
from jax.experimental.pallas import tpu as pltpu
from jax.experimental.pallas import tpu_sc as plsc

N = 10000
E = 320000
D = 128
G = 64
BN_EPS = 1e-5

NC, NS = 2, 16
NW = NC * NS
CHUNK = 128
C0 = 76
C1 = 81
SEG = 32
MAXC = 96
E_CAP0 = NS * C0 * CHUNK
E_TOT = NS * (C0 + C1) * CHUNK
NP = 10112
NP0 = N
RPT = NP // NS

_mesh = plsc.VectorSubcoreMesh(core_axis_name="c", subcore_axis_name="s")


def _zero_tile_buf(buf, nrows, ncols):
    zv = jnp.zeros((16,), jnp.float32)

    def body(r, _):
        for c in range(ncols // 16):
            buf[r, pl.ds(c * 16, 16)] = zv
        return 0

    lax.fori_loop(0, nrows, body, 0)


def _zero_acc_slice(buf, acc_sh, base):
    done = 0
    while done < RPT:
        nr = min(CHUNK, RPT - done)
        pltpu.sync_copy(buf.at[pl.ds(0, nr)],
                        acc_sh.at[pl.ds(base + done, nr)])
        done += nr


@functools.partial(
    pl.kernel,
    out_type=jax.ShapeDtypeStruct((NC, NP, D), jnp.float32),
    mesh=_mesh,
    scratch_types=[
        pltpu.VMEM_SHARED((NP, D), jnp.float32),
        pltpu.VMEM((SEG, CHUNK), jnp.int32),
        pltpu.VMEM((CHUNK, D), jnp.float32),
        pltpu.SemaphoreType.DMA,
    ],
)
def _sc_deg(dst_hbm, out_hbm, acc_sh, dst_v, buf_v, sem):
    cid = lax.axis_index("c")
    sid = lax.axis_index("s")
    wid = cid * NS + sid
    myc = jnp.where(cid == 0, C0, C1)

    _zero_tile_buf(buf_v, CHUNK, D)
    base = sid * RPT
    _zero_acc_slice(buf_v, acc_sh, base)

    ones = jnp.ones((16,), jnp.float32)

    def fill(r, _):
        for c in range(D // 16):
            buf_v[r, pl.ds(c * 16, 16)] = ones
        return 0

    lax.fori_loop(0, CHUNK, fill, 0)
    plsc.subcore_barrier()

    DEPTH = 4

    def issue(j):
        pltpu.async_copy(buf_v, acc_sh.at[dst_v.at[j]], sem, add=True)

    def drain(j):
        pltpu.make_async_copy(buf_v, acc_sh.at[dst_v.at[j]], sem).wait()

    def seg_body(s, _):
        nj = jnp.minimum(SEG, myc - s * SEG)

        pltpu.sync_copy(dst_hbm.at[wid].at[pl.ds(s * SEG, SEG)], dst_v)

        def prime(j, _):
            issue(j)
            return 0

        np_ = jnp.minimum(nj, DEPTH)
        lax.fori_loop(0, np_, prime, 0)

        def chunk(j, _):
            issue(j)
            drain(j - DEPTH)
            return 0

        lax.fori_loop(np_, nj, chunk, 0)

        def dr(j, _):
            drain(j)
            return 0

        lax.fori_loop(jnp.maximum(nj - DEPTH, 0), nj, dr, 0)
        return 0

    nseg = (myc + SEG - 1) // SEG
    lax.fori_loop(0, nseg, seg_body, 0)

    plsc.subcore_barrier()
    pltpu.sync_copy(acc_sh.at[pl.ds(base, RPT)],
                    out_hbm.at[cid].at[pl.ds(base, RPT)])


@functools.partial(
    pl.kernel,
    out_type=jax.ShapeDtypeStruct((NC, NP, D), jnp.float32),
    mesh=_mesh,
    scratch_types=[
        pltpu.VMEM_SHARED((NP, D), jnp.float32),
        pltpu.VMEM((SEG, CHUNK), jnp.int32),
        pltpu.VMEM((SEG, CHUNK), jnp.int32),
        pltpu.VMEM((CHUNK, D), jnp.float32),
        pltpu.VMEM((CHUNK, D), jnp.float32),
        pltpu.SemaphoreType.DMA,
        pltpu.SemaphoreType.DMA,
        pltpu.SemaphoreType.DMA,
    ],
)
def _sc_prop(hp_hbm, src_hbm, dst_hbm, out_hbm,
             acc_sh, src_v, dst_v, buf0, buf1, gsem, s0, s1):
    cid = lax.axis_index("c")
    sid = lax.axis_index("s")
    wid = cid * NS + sid
    myc = jnp.where(cid == 0, C0, C1)
    bufs = (buf0, buf1)
    ssem = (s0, s1)

    _zero_tile_buf(buf0, CHUNK, D)
    base = sid * RPT
    _zero_acc_slice(buf0, acc_sh, base)
    plsc.subcore_barrier()

    def gather(j, b):
        pltpu.async_copy(hp_hbm.at[src_v.at[j]], bufs[b], gsem).wait()

    def s_issue(j, b):
        pltpu.async_copy(bufs[b], acc_sh.at[dst_v.at[j]], ssem[b], add=True)

    def s_wait(j, b):
        pltpu.make_async_copy(bufs[b], acc_sh.at[dst_v.at[j]], ssem[b]).wait()

    def seg_body(s, _):
        nj = jnp.minimum(SEG, myc - s * SEG)
        pltpu.sync_copy(src_hbm.at[wid].at[pl.ds(s * SEG, SEG)], src_v)
        pltpu.sync_copy(dst_hbm.at[wid].at[pl.ds(s * SEG, SEG)], dst_v)

        def pair(p, _):
            j0 = 2 * p
            j1 = j0 + 1

            @pl.when(p > 0)
            def _():
                s_wait(j0 - 2, 0)

            gather(j0, 0)
            s_issue(j0, 0)

            @pl.when(j1 < nj)
            def _():
                @pl.when(p > 0)
                def _():
                    s_wait(j1 - 2, 1)

                gather(j1, 1)
                s_issue(j1, 1)

            return 0

        npair = (nj + 1) // 2
        lax.fori_loop(0, npair, pair, 0)

        s_wait(0, 0)

        @pl.when(nj >= 2)
        def _():
            s_wait(0, 1)

        return 0

    nseg = (myc + SEG - 1) // SEG
    lax.fori_loop(0, nseg, seg_body, 0)

    plsc.subcore_barrier()
    pltpu.sync_copy(acc_sh.at[pl.ds(base, RPT)],
                    out_hbm.at[cid].at[pl.ds(base, RPT)])


def _tc_pre_body(deg2_ref, x_ref, w1_ref, dinv_ref, h1p_ref):
    deg = deg2_ref[0] + deg2_ref[1]
    dinv_b = jax.lax.rsqrt(deg + 1.0)
    dinv_ref[...] = dinv_b
    h = jnp.dot(x_ref[...], w1_ref[...].T, preferred_element_type=jnp.float32)
    h1p_ref[...] = dinv_b * h


def _tc_mid_body(acc_ref, hp_ref, dinv_ref, b_ref, g_ref, be_ref, wn_ref,
                 out_ref):
    dinv_b = dinv_ref[...]
    z = dinv_b * (acc_ref[0] + acc_ref[1] + hp_ref[...]) + b_ref[...]
    z = z * (g_ref[...] * jax.lax.rsqrt(jnp.float32(1.0 + BN_EPS))) + be_ref[...]
    z = jnp.maximum(z, 0.0)
    h = jnp.dot(z, wn_ref[...].T, preferred_element_type=jnp.float32)
    rows = lax.broadcasted_iota(jnp.int32, (NP, D), 0)
    out_ref[...] = jnp.where(rows < N, dinv_b * h, 0.0)


def _tc_final_body(acc_ref, hp_ref, dinv_ref, b_ref, batch_ref,
                   lw1_ref, lb1_ref, lw2b_ref, lb2b_ref, out_ref):
    h3 = dinv_ref[...] * (acc_ref[0] + acc_ref[1] + hp_ref[...]) + b_ref[...]
    h3 = h3[:N]
    seg = lax.broadcasted_iota(jnp.int32, (G, N), 0)
    bat = jnp.broadcast_to(batch_ref[...][None, :], (G, N))
    mask = (seg == bat).astype(jnp.float32)
    sums = jnp.dot(mask, h3, preferred_element_type=jnp.float32)
    cnt = jnp.dot(mask, jnp.full((N, D), 1.0, jnp.float32),
                  preferred_element_type=jnp.float32)
    pooled = sums / jnp.maximum(cnt, 1.0)
    hm = jnp.dot(pooled, lw1_ref[...].T, preferred_element_type=jnp.float32)
    hm = jnp.maximum(hm + lb1_ref[...], 0.0)
    o = jnp.dot(hm, lw2b_ref[...], preferred_element_type=jnp.float32)
    out_ref[...] = o + lb2b_ref[...]


_tc_pre = pl.pallas_call(
    _tc_pre_body,
    out_shape=(jax.ShapeDtypeStruct((NP, D), jnp.float32),
               jax.ShapeDtypeStruct((NP, D), jnp.float32)),
)

_tc_mid = pl.pallas_call(
    _tc_mid_body,
    out_shape=jax.ShapeDtypeStruct((NP, D), jnp.float32),
)

_tc_final = pl.pallas_call(
    _tc_final_body,
    out_shape=jax.ShapeDtypeStruct((G, D), jnp.float32),
)


def _edge_layout(idx):
    ec = jnp.concatenate([idx, jnp.full((E_TOT - E,), NP0, jnp.int32)])
    p0 = ec[:E_CAP0].reshape(NS, C0, CHUNK)
    p0 = jnp.concatenate(
        [p0, jnp.full((NS, MAXC - C0, CHUNK), NP0, jnp.int32)], axis=1)
    p1 = ec[E_CAP0:].reshape(NS, C1, CHUNK)
    p1 = jnp.concatenate(
        [p1, jnp.full((NS, MAXC - C1, CHUNK), NP0, jnp.int32)], axis=1)
    return jnp.concatenate([p0, p1], axis=0)


def kernel(x, edge_index, batch, W1, b1, g1, be1, W2, b2, g2, be2, W3, b3,
           lw1, lb1, lw2, lb2):
    src_p = _edge_layout(edge_index[0])
    dst_p = _edge_layout(edge_index[1])

    x_p = jnp.zeros((NP, D), jnp.float32).at[:N].set(x)

    deg2 = _sc_deg(dst_p)
    dinv_b, h1p = _tc_pre(deg2, x_p, W1)
    acc1 = _sc_prop(h1p, src_p, dst_p)
    h2p = _tc_mid(acc1, h1p, dinv_b, b1, g1, be1, W2)
    acc2 = _sc_prop(h2p, src_p, dst_p)
    h3p = _tc_mid(acc2, h2p, dinv_b, b2, g2, be2, W3)
    acc3 = _sc_prop(h3p, src_p, dst_p)
    lw2b = jnp.broadcast_to(lw2.T, (G, D))
    lb2b = jnp.broadcast_to(lb2, (D,))
    out128 = _tc_final(acc3, h3p, dinv_b, b3, batch, lw1, lb1, lw2b, lb2b)
    return out128[:, :1]


# speedup vs baseline: 1.2751x; 1.0150x over previous
import functools
import jax
import jax.numpy as jnp
from jax import lax
from jax.experimental import pallas as pl
from jax.experimental.pallas import tpu as pltpu
from jax.experimental.pallas import tpu_sc as plsc

N = 10000
E = 320000
D = 128
G = 64
BN_EPS = 1e-5

NC, NS = 2, 16
NW = NC * NS
CHUNK = 128
C0 = 79
C1 = 78
SEG = 32
MAXC = 96
E_CAP0 = NS * C0 * CHUNK
E_TOT = NS * (C0 + C1) * CHUNK
NP = 10112
NP0 = N
RPT = NP // NS

_mesh = plsc.VectorSubcoreMesh(core_axis_name="c", subcore_axis_name="s")


def _zero_tile_buf(buf, nrows, ncols):
    zv = jnp.zeros((16,), jnp.float32)

    def body(r, _):
        for c in range(ncols // 16):
            buf[r, pl.ds(c * 16, 16)] = zv
        return 0

    lax.fori_loop(0, nrows, body, 0)


def _zero_acc_slice(buf, acc_sh, base):
    done = 0
    while done < RPT:
        nr = min(CHUNK, RPT - done)
        pltpu.sync_copy(buf.at[pl.ds(0, nr)],
                        acc_sh.at[pl.ds(base + done, nr)])
        done += nr


@functools.partial(
    pl.kernel,
    out_type=jax.ShapeDtypeStruct((NC, NP, D), jnp.float32),
    mesh=_mesh,
    scratch_types=[
        pltpu.VMEM_SHARED((NP, D), jnp.float32),
        pltpu.VMEM((SEG, CHUNK), jnp.int32),
        pltpu.VMEM((CHUNK, D), jnp.float32),
        pltpu.SemaphoreType.DMA,
    ],
)
def _sc_deg(dst_hbm, out_hbm, acc_sh, dst_v, buf_v, sem):
    cid = lax.axis_index("c")
    sid = lax.axis_index("s")
    wid = cid * NS + sid
    myc = jnp.where(cid == 0, C0, C1)

    _zero_tile_buf(buf_v, CHUNK, D)
    base = sid * RPT
    _zero_acc_slice(buf_v, acc_sh, base)

    ones = jnp.ones((16,), jnp.float32)

    def fill(r, _):
        for c in range(D // 16):
            buf_v[r, pl.ds(c * 16, 16)] = ones
        return 0

    lax.fori_loop(0, CHUNK, fill, 0)
    plsc.subcore_barrier()

    DEPTH = 4

    def issue(j):
        pltpu.async_copy(buf_v, acc_sh.at[dst_v.at[j]], sem, add=True)

    def drain(j):
        pltpu.make_async_copy(buf_v, acc_sh.at[dst_v.at[j]], sem).wait()

    def seg_body(s, _):
        nj = jnp.minimum(SEG, myc - s * SEG)

        pltpu.sync_copy(dst_hbm.at[wid].at[pl.ds(s * SEG, SEG)], dst_v)

        def prime(j, _):
            issue(j)
            return 0

        np_ = jnp.minimum(nj, DEPTH)
        lax.fori_loop(0, np_, prime, 0)

        def chunk(j, _):
            issue(j)
            drain(j - DEPTH)
            return 0

        lax.fori_loop(np_, nj, chunk, 0)

        def dr(j, _):
            drain(j)
            return 0

        lax.fori_loop(jnp.maximum(nj - DEPTH, 0), nj, dr, 0)
        return 0

    nseg = (myc + SEG - 1) // SEG
    lax.fori_loop(0, nseg, seg_body, 0)

    plsc.subcore_barrier()
    pltpu.sync_copy(acc_sh.at[pl.ds(base, RPT)],
                    out_hbm.at[cid].at[pl.ds(base, RPT)])


@functools.partial(
    pl.kernel,
    out_type=jax.ShapeDtypeStruct((NC, NP, D), jnp.float32),
    mesh=_mesh,
    scratch_types=[
        pltpu.VMEM_SHARED((NP, D), jnp.float32),
        pltpu.VMEM((SEG, CHUNK), jnp.int32),
        pltpu.VMEM((SEG, CHUNK), jnp.int32),
        pltpu.VMEM((CHUNK, D), jnp.float32),
        pltpu.VMEM((CHUNK, D), jnp.float32),
        pltpu.SemaphoreType.DMA,
        pltpu.SemaphoreType.DMA,
        pltpu.SemaphoreType.DMA,
    ],
)
def _sc_prop(hp_hbm, src_hbm, dst_hbm, out_hbm,
             acc_sh, src_v, dst_v, buf0, buf1, gsem, s0, s1):
    cid = lax.axis_index("c")
    sid = lax.axis_index("s")
    wid = cid * NS + sid
    myc = jnp.where(cid == 0, C0, C1)
    bufs = (buf0, buf1)
    ssem = (s0, s1)

    _zero_tile_buf(buf0, CHUNK, D)
    base = sid * RPT
    _zero_acc_slice(buf0, acc_sh, base)
    plsc.subcore_barrier()

    def gather(j, b):
        pltpu.async_copy(hp_hbm.at[src_v.at[j]], bufs[b], gsem).wait()

    def s_issue(j, b):
        pltpu.async_copy(bufs[b], acc_sh.at[dst_v.at[j]], ssem[b], add=True)

    def s_wait(j, b):
        pltpu.make_async_copy(bufs[b], acc_sh.at[dst_v.at[j]], ssem[b]).wait()

    def seg_body(s, _):
        nj = jnp.minimum(SEG, myc - s * SEG)
        pltpu.sync_copy(src_hbm.at[wid].at[pl.ds(s * SEG, SEG)], src_v)
        pltpu.sync_copy(dst_hbm.at[wid].at[pl.ds(s * SEG, SEG)], dst_v)

        def pair(p, _):
            j0 = 2 * p
            j1 = j0 + 1

            @pl.when(p > 0)
            def _():
                s_wait(j0 - 2, 0)

            gather(j0, 0)
            s_issue(j0, 0)

            @pl.when(j1 < nj)
            def _():
                @pl.when(p > 0)
                def _():
                    s_wait(j1 - 2, 1)

                gather(j1, 1)
                s_issue(j1, 1)

            return 0

        npair = (nj + 1) // 2
        lax.fori_loop(0, npair, pair, 0)

        s_wait(0, 0)

        @pl.when(nj >= 2)
        def _():
            s_wait(0, 1)

        return 0

    nseg = (myc + SEG - 1) // SEG
    lax.fori_loop(0, nseg, seg_body, 0)

    plsc.subcore_barrier()
    pltpu.sync_copy(acc_sh.at[pl.ds(base, RPT)],
                    out_hbm.at[cid].at[pl.ds(base, RPT)])


def _tc_pre_body(deg2_ref, x_ref, w1_ref, dinv_ref, h1p_ref):
    deg = deg2_ref[0] + deg2_ref[1]
    dinv_b = jax.lax.rsqrt(deg + 1.0)
    dinv_ref[...] = dinv_b
    h = jnp.dot(x_ref[...], w1_ref[...].T, preferred_element_type=jnp.float32)
    h1p_ref[...] = dinv_b * h


def _tc_mid_body(acc_ref, hp_ref, dinv_ref, b_ref, g_ref, be_ref, wn_ref,
                 out_ref):
    dinv_b = dinv_ref[...]
    z = dinv_b * (acc_ref[0] + acc_ref[1] + hp_ref[...]) + b_ref[...]
    z = z * (g_ref[...] * jax.lax.rsqrt(jnp.float32(1.0 + BN_EPS))) + be_ref[...]
    z = jnp.maximum(z, 0.0)
    h = jnp.dot(z, wn_ref[...].T, preferred_element_type=jnp.float32)
    rows = lax.broadcasted_iota(jnp.int32, (NP, D), 0)
    out_ref[...] = jnp.where(rows < N, dinv_b * h, 0.0)


def _tc_final_body(acc_ref, hp_ref, dinv_ref, b_ref, batch_ref,
                   lw1_ref, lb1_ref, lw2b_ref, lb2b_ref, out_ref):
    h3 = dinv_ref[...] * (acc_ref[0] + acc_ref[1] + hp_ref[...]) + b_ref[...]
    h3 = h3[:N]
    seg = lax.broadcasted_iota(jnp.int32, (G, N), 0)
    bat = jnp.broadcast_to(batch_ref[...][None, :], (G, N))
    mask = (seg == bat).astype(jnp.float32)
    sums = jnp.dot(mask, h3, preferred_element_type=jnp.float32)
    cnt = jnp.dot(mask, jnp.full((N, D), 1.0, jnp.float32),
                  preferred_element_type=jnp.float32)
    pooled = sums / jnp.maximum(cnt, 1.0)
    hm = jnp.dot(pooled, lw1_ref[...].T, preferred_element_type=jnp.float32)
    hm = jnp.maximum(hm + lb1_ref[...], 0.0)
    o = jnp.dot(hm, lw2b_ref[...], preferred_element_type=jnp.float32)
    out_ref[...] = o + lb2b_ref[...]


_tc_pre = pl.pallas_call(
    _tc_pre_body,
    out_shape=(jax.ShapeDtypeStruct((NP, D), jnp.float32),
               jax.ShapeDtypeStruct((NP, D), jnp.float32)),
)

_tc_mid = pl.pallas_call(
    _tc_mid_body,
    out_shape=jax.ShapeDtypeStruct((NP, D), jnp.float32),
)

_tc_final = pl.pallas_call(
    _tc_final_body,
    out_shape=jax.ShapeDtypeStruct((G, D), jnp.float32),
)


def _edge_layout(idx):
    ec = jnp.concatenate([idx, jnp.full((E_TOT - E,), NP0, jnp.int32)])
    p0 = ec[:E_CAP0].reshape(NS, C0, CHUNK)
    p0 = jnp.concatenate(
        [p0, jnp.full((NS, MAXC - C0, CHUNK), NP0, jnp.int32)], axis=1)
    p1 = ec[E_CAP0:].reshape(NS, C1, CHUNK)
    p1 = jnp.concatenate(
        [p1, jnp.full((NS, MAXC - C1, CHUNK), NP0, jnp.int32)], axis=1)
    return jnp.concatenate([p0, p1], axis=0)


def kernel(x, edge_index, batch, W1, b1, g1, be1, W2, b2, g2, be2, W3, b3,
           lw1, lb1, lw2, lb2):
    src_p = _edge_layout(edge_index[0])
    dst_p = _edge_layout(edge_index[1])

    x_p = jnp.zeros((NP, D), jnp.float32).at[:N].set(x)

    deg2 = _sc_deg(dst_p)
    dinv_b, h1p = _tc_pre(deg2, x_p, W1)
    acc1 = _sc_prop(h1p, src_p, dst_p)
    h2p = _tc_mid(acc1, h1p, dinv_b, b1, g1, be1, W2)
    acc2 = _sc_prop(h2p, src_p, dst_p)
    h3p = _tc_mid(acc2, h2p, dinv_b, b2, g2, be2, W3)
    acc3 = _sc_prop(h3p, src_p, dst_p)
    lw2b = jnp.broadcast_to(lw2.T, (G, D))
    lb2b = jnp.broadcast_to(lb2, (D,))
    out128 = _tc_final(acc3, h3p, dinv_b, b3, batch, lw1, lb1, lw2b, lb2b)
    return out128[:, :1]
